# Initial kernel scaffold; baseline (speedup 1.0000x reference)
import jax
import jax.numpy as jnp
from jax.experimental import pallas as pl


def kernel(x, A, B, C, D):
    raise NotImplementedError("write your pallas kernel here")


# speedup vs baseline: 3.0546x; 3.0546x over previous
import jax
import jax.numpy as jnp
from jax.experimental import pallas as pl
from jax.experimental.pallas import tpu as pltpu

TAU = 2.0
V_TH = 1.0

S_TILE = 128


def _ssm_kernel(x_ref, At_ref, Bt_ref, Ct_ref, D_ref, out_ref):
    T = x_ref.shape[1]
    s_tile = x_ref.shape[2]
    d_state = At_ref.shape[0]
    At = At_ref[...]
    Bt = Bt_ref[...]
    Ct = Ct_ref[...]
    D = D_ref[...]

    h = jnp.zeros((s_tile, d_state), dtype=jnp.float32)
    vs = jnp.zeros((s_tile, d_state), dtype=jnp.float32)
    vo = jnp.zeros((s_tile, x_ref.shape[3]), dtype=jnp.float32)

    for t in range(T):
        xt = x_ref[0, t]
        su = (jnp.dot(h, At, preferred_element_type=jnp.float32)
              + jnp.dot(xt, Bt, preferred_element_type=jnp.float32))
        vs = vs + (su - vs) / TAU
        s = (vs >= V_TH).astype(jnp.float32)
        vs = vs * (1.0 - s)
        ou = jnp.dot(s, Ct, preferred_element_type=jnp.float32) + xt + D
        vo = vo + (ou - vo) / TAU
        so = (vo >= V_TH).astype(jnp.float32)
        vo = vo * (1.0 - so)
        out_ref[0, t] = so
        h = s


def kernel(x, A, B, C, D):
    Bsz, T, S, d_model = x.shape
    d_state = A.shape[0]
    At = A.T
    Bt = B.T
    Ct = C.T
    D2 = D.reshape(1, d_model)

    grid = (Bsz, S // S_TILE)
    return pl.pallas_call(
        _ssm_kernel,
        grid=grid,
        in_specs=[
            pl.BlockSpec((1, T, S_TILE, d_model), lambda b, s: (b, 0, s, 0)),
            pl.BlockSpec((d_state, d_state), lambda b, s: (0, 0)),
            pl.BlockSpec((d_model, d_state), lambda b, s: (0, 0)),
            pl.BlockSpec((d_state, d_model), lambda b, s: (0, 0)),
            pl.BlockSpec((1, d_model), lambda b, s: (0, 0)),
        ],
        out_specs=pl.BlockSpec((1, T, S_TILE, d_model), lambda b, s: (b, 0, s, 0)),
        out_shape=jax.ShapeDtypeStruct((Bsz, T, S, d_model), jnp.float32),
        compiler_params=pltpu.CompilerParams(
            dimension_semantics=("parallel", "parallel"),
        ),
    )(x, At, Bt, Ct, D2)


# speedup vs baseline: 4.0032x; 1.3106x over previous
import jax
import jax.numpy as jnp
from jax.experimental import pallas as pl
from jax.experimental.pallas import tpu as pltpu

TAU = 2.0
V_TH = 1.0

S_TILE = 256


def _ssm_kernel(x_ref, At_ref, Bt_ref, Ct_ref, D_ref, out_ref):
    T = x_ref.shape[1]
    s_tile = x_ref.shape[2]
    d_state = At_ref.shape[0]
    At = At_ref[...]
    Bt = Bt_ref[...]
    Ct = Ct_ref[...]
    D = D_ref[...]

    h = jnp.zeros((s_tile, d_state), dtype=jnp.float32)
    vs = jnp.zeros((s_tile, d_state), dtype=jnp.float32)
    vo = jnp.zeros((s_tile, x_ref.shape[3]), dtype=jnp.float32)

    for t in range(T):
        xt = x_ref[0, t]
        su = (jnp.dot(h, At, preferred_element_type=jnp.float32)
              + jnp.dot(xt, Bt, preferred_element_type=jnp.float32))
        vs = vs + (su - vs) / TAU
        s = (vs >= V_TH).astype(jnp.float32)
        vs = vs * (1.0 - s)
        ou = jnp.dot(s, Ct, preferred_element_type=jnp.float32) + xt + D
        vo = vo + (ou - vo) / TAU
        so = (vo >= V_TH).astype(jnp.float32)
        vo = vo * (1.0 - so)
        out_ref[0, t] = so
        h = s


def kernel(x, A, B, C, D):
    Bsz, T, S, d_model = x.shape
    d_state = A.shape[0]
    At = A.T
    Bt = B.T
    Ct = C.T
    D2 = D.reshape(1, d_model)

    grid = (Bsz, S // S_TILE)
    return pl.pallas_call(
        _ssm_kernel,
        grid=grid,
        in_specs=[
            pl.BlockSpec((1, T, S_TILE, d_model), lambda b, s: (b, 0, s, 0)),
            pl.BlockSpec((d_state, d_state), lambda b, s: (0, 0)),
            pl.BlockSpec((d_model, d_state), lambda b, s: (0, 0)),
            pl.BlockSpec((d_state, d_model), lambda b, s: (0, 0)),
            pl.BlockSpec((1, d_model), lambda b, s: (0, 0)),
        ],
        out_specs=pl.BlockSpec((1, T, S_TILE, d_model), lambda b, s: (b, 0, s, 0)),
        out_shape=jax.ShapeDtypeStruct((Bsz, T, S, d_model), jnp.float32),
        compiler_params=pltpu.CompilerParams(
            dimension_semantics=("parallel", "parallel"),
            vmem_limit_bytes=56 * 1024 * 1024,
        ),
    )(x, At, Bt, Ct, D2)
